# Initial kernel scaffold; baseline (speedup 1.0000x reference)
import jax
import jax.numpy as jnp
from jax.experimental import pallas as pl


def kernel(img, W0, W1, W2, b0, b1, b2, g_logits):
    raise NotImplementedError("write your pallas kernel here")


# speedup vs baseline: 4.0487x; 4.0487x over previous
import functools

import jax
import jax.numpy as jnp
from jax.experimental import pallas as pl
from jax.experimental.pallas import tpu as pltpu

N_TASKS = 4
BLOCKS = 3
D = 768
N_TOK = 4096
TILE_N = 512


def _gating_kernel(g_ref, img_ref, w0_ref, w1_ref, w2_ref,
                   b0_ref, b1_ref, b2_ref, out_ref):
    x = img_ref[:]
    blocks = [
        jnp.dot(x, w0_ref[:], preferred_element_type=jnp.float32) + b0_ref[:],
        jnp.dot(x, w1_ref[:], preferred_element_type=jnp.float32) + b1_ref[:],
        jnp.dot(x, w2_ref[:], preferred_element_type=jnp.float32) + b2_ref[:],
    ]
    for t in range(N_TASKS):
        g = [g_ref[t, b] for b in range(BLOCKS)]
        m = jnp.maximum(jnp.maximum(g[0], g[1]), g[2])
        e = [jnp.exp(gi - m) for gi in g]
        s = e[0] + e[1] + e[2]
        acc = blocks[0] * (e[0] / s)
        acc += blocks[1] * (e[1] / s)
        acc += blocks[2] * (e[2] / s)
        out_ref[t] = acc


@functools.partial(jax.jit, static_argnames=())
def kernel(img, W0, W1, W2, b0, b1, b2, g_logits):
    grid = (N_TOK // TILE_N,)
    out = pl.pallas_call(
        _gating_kernel,
        grid=grid,
        in_specs=[
            pl.BlockSpec(memory_space=pltpu.SMEM),
            pl.BlockSpec((TILE_N, D), lambda i: (i, 0)),
            pl.BlockSpec((D, D), lambda i: (0, 0)),
            pl.BlockSpec((D, D), lambda i: (0, 0)),
            pl.BlockSpec((D, D), lambda i: (0, 0)),
            pl.BlockSpec((1, D), lambda i: (0, 0)),
            pl.BlockSpec((1, D), lambda i: (0, 0)),
            pl.BlockSpec((1, D), lambda i: (0, 0)),
        ],
        out_specs=pl.BlockSpec((N_TASKS, TILE_N, D), lambda i: (0, i, 0)),
        out_shape=jax.ShapeDtypeStruct((N_TASKS, N_TOK, D), jnp.float32),
    )(g_logits, img, W0, W1, W2,
      b0.reshape(1, D), b1.reshape(1, D), b2.reshape(1, D))
    return out
